# baseline (speedup 1.0000x reference)
import functools

import jax
import jax.numpy as jnp
from jax import lax
from jax.experimental import pallas as pl
from jax.experimental.pallas import tpu as pltpu
from jax.experimental.pallas import tpu_sc as plsc

B, T, F = 1024, 50, 26
VOCAB, D = 1000, 16
N = B * T * F
NC, NS = 2, 16
NW = NC * NS
PER_W = N // NW
CHUNK = 2080
NCHUNK = PER_W // CHUNK
L = 16


def _make_gather():
    mesh = plsc.VectorSubcoreMesh(core_axis_name="c", subcore_axis_name="s")

    @functools.partial(
        pl.kernel,
        mesh=mesh,
        out_type=jax.ShapeDtypeStruct((N, D), jnp.float32),
        scratch_types=[
            pltpu.VMEM((VOCAB, D), jnp.float32),
            pltpu.VMEM((2, CHUNK), jnp.int32),
            pltpu.VMEM((2, CHUNK, D), jnp.float32),
            pltpu.SemaphoreType.DMA,
            pltpu.SemaphoreType.DMA,
            pltpu.SemaphoreType.DMA,
            pltpu.SemaphoreType.DMA,
        ],
        compiler_params=pltpu.CompilerParams(use_tc_tiling_on_sc=False,
                                             needs_layout_passes=False),
    )
    def gather_kernel(idx_hbm, table_hbm, out_hbm, table_v, idx_v, rows_v,
                      isem0, isem1, wsem0, wsem1):
        wid = lax.axis_index("s") * NC + lax.axis_index("c")
        base = wid * PER_W
        isems = (isem0, isem1)
        wsems = (wsem0, wsem1)

        tcopy = pltpu.async_copy(table_hbm, table_v, wsems[0])

        icopies = [None, None]
        for g in range(2):
            icopies[g] = pltpu.async_copy(
                idx_hbm.at[pl.ds(base + g * CHUNK, CHUNK)],
                idx_v.at[g], isems[g])

        tcopy.wait()
        lane = lax.iota(jnp.int32, L)

        def gather_chunk(rows_b, idx_b):
            def body(j, carry):
                iv = idx_b[pl.ds(j * L, L)]
                rid = j * L + lane
                for c in range(D):
                    cs = jnp.full((L,), c, jnp.int32)
                    col = plsc.load_gather(table_v, [iv, cs])
                    plsc.store_scatter(rows_b, [rid, cs], col)
                return carry

            lax.fori_loop(0, CHUNK // L, body, 0)

        wcopies = [None, None]
        for g in range(NCHUNK):
            b = g % 2
            off = base + g * CHUNK
            if wcopies[b] is not None:
                wcopies[b].wait()
            icopies[b].wait()
            gather_chunk(rows_v.at[b], idx_v.at[b])
            if g + 2 < NCHUNK:
                icopies[b] = pltpu.async_copy(
                    idx_hbm.at[pl.ds(base + (g + 2) * CHUNK, CHUNK)],
                    idx_v.at[b], isems[b])
            wcopies[b] = pltpu.async_copy(
                rows_v.at[b], out_hbm.at[pl.ds(off, CHUNK)], wsems[b])

        for w in wcopies:
            if w is not None:
                w.wait()

    return gather_kernel


_gather = _make_gather()


def kernel(x, vec_of_char):
    idx = x.reshape(N)
    out = _gather(idx, vec_of_char)
    return out.reshape(B, T, F * D)


# speedup vs baseline: 1.2584x; 1.2584x over previous
import functools

import jax
import jax.numpy as jnp
from jax import lax
from jax.experimental import pallas as pl
from jax.experimental.pallas import tpu as pltpu
from jax.experimental.pallas import tpu_sc as plsc

B, T, F = 1024, 50, 26
VOCAB, D = 1000, 16
N = B * T * F
NC, NS = 2, 16
NW = NC * NS
PER_W = N // NW
CHUNK = 2080
NCHUNK = PER_W // CHUNK
L = 16


def _make_gather():
    mesh = plsc.VectorSubcoreMesh(core_axis_name="c", subcore_axis_name="s")

    @functools.partial(
        pl.kernel,
        mesh=mesh,
        out_type=jax.ShapeDtypeStruct((N, D), jnp.float32),
        scratch_types=[
            pltpu.VMEM((VOCAB, D), jnp.float32),
            pltpu.VMEM((2, CHUNK), jnp.int32),
            pltpu.VMEM((2, CHUNK, D), jnp.float32),
            pltpu.SemaphoreType.DMA,
            pltpu.SemaphoreType.DMA,
            pltpu.SemaphoreType.DMA,
            pltpu.SemaphoreType.DMA,
            pltpu.SemaphoreType.DMA,
        ],
        compiler_params=pltpu.CompilerParams(use_tc_tiling_on_sc=False,
                                             needs_layout_passes=False),
    )
    def gather_kernel(idx_hbm, table_hbm, out_hbm, table_v, idx_v, rows_v,
                      tsem, isem0, isem1, wsem0, wsem1):
        wid = lax.axis_index("s") * NC + lax.axis_index("c")
        base = wid * PER_W
        isems = (isem0, isem1)
        wsems = (wsem0, wsem1)
        lane = lax.iota(jnp.int32, L)

        tcopy = pltpu.async_copy(table_hbm, table_v, tsem)
        icopies = [
            pltpu.async_copy(idx_hbm.at[pl.ds(base + g * CHUNK, CHUNK)],
                             idx_v.at[g], isems[g])
            for g in range(2)
        ]
        tcopy.wait()

        def gather_chunk(rows_b, idx_b):
            @plsc.parallel_loop(0, CHUNK // L, unroll=4)
            def body(j):
                iv = idx_b[pl.ds(j * L, L)]
                rid = j * L + lane
                for c in range(D):
                    cs = jnp.full((L,), c, jnp.int32)
                    col = plsc.load_gather(table_v, [iv, cs])
                    plsc.store_scatter(rows_b, [rid, cs], col)

        for b in range(2):
            icopies[b].wait()
            gather_chunk(rows_v.at[b], idx_v.at[b])
            pltpu.async_copy(idx_hbm.at[pl.ds(base + (b + 2) * CHUNK, CHUNK)],
                             idx_v.at[b], isems[b])
            pltpu.async_copy(rows_v.at[b],
                             out_hbm.at[pl.ds(base + b * CHUNK, CHUNK)],
                             wsems[b])

        def pair(i, carry):
            for b in range(2):
                g = 2 * i + b
                off = base + g * CHUNK
                pltpu.make_async_copy(
                    rows_v.at[b], out_hbm.at[pl.ds(off, CHUNK)],
                    wsems[b]).wait()
                pltpu.make_async_copy(
                    idx_hbm.at[pl.ds(off, CHUNK)], idx_v.at[b],
                    isems[b]).wait()
                gather_chunk(rows_v.at[b], idx_v.at[b])
                off2 = base + lax.rem(g + 2, NCHUNK) * CHUNK
                pltpu.async_copy(idx_hbm.at[pl.ds(off2, CHUNK)],
                                 idx_v.at[b], isems[b])
                pltpu.async_copy(rows_v.at[b],
                                 out_hbm.at[pl.ds(off, CHUNK)], wsems[b])
            return carry

        lax.fori_loop(1, NCHUNK // 2, pair, 0)

        for b in range(2):
            pltpu.make_async_copy(
                rows_v.at[b], out_hbm.at[pl.ds(base, CHUNK)],
                wsems[b]).wait()
            pltpu.make_async_copy(
                idx_hbm.at[pl.ds(base, CHUNK)], idx_v.at[b],
                isems[b]).wait()

    return gather_kernel


_gather = _make_gather()


def kernel(x, vec_of_char):
    idx = x.reshape(N)
    out = _gather(idx, vec_of_char)
    return out.reshape(B, T, F * D)


# speedup vs baseline: 1.4355x; 1.1408x over previous
import functools

import jax
import jax.numpy as jnp
from jax import lax
from jax.experimental import pallas as pl
from jax.experimental.pallas import tpu as pltpu
from jax.experimental.pallas import tpu_sc as plsc

B, T, F = 1024, 50, 26
VOCAB, D = 1000, 16
N = B * T * F
NC, NS = 2, 16
NW = NC * NS
PER_W = N // NW
CHUNK = 2080
NCHUNK = PER_W // CHUNK
L = 16
RB = CHUNK * D


def _make_gather():
    mesh = plsc.VectorSubcoreMesh(core_axis_name="c", subcore_axis_name="s")

    @functools.partial(
        pl.kernel,
        mesh=mesh,
        out_type=jax.ShapeDtypeStruct((N * D,), jnp.float32),
        scratch_types=[
            pltpu.VMEM((VOCAB * D,), jnp.float32),
            pltpu.VMEM((2, CHUNK), jnp.int32),
            pltpu.VMEM((2, RB), jnp.float32),
            pltpu.SemaphoreType.DMA,
            pltpu.SemaphoreType.DMA,
            pltpu.SemaphoreType.DMA,
            pltpu.SemaphoreType.DMA,
            pltpu.SemaphoreType.DMA,
        ],
        compiler_params=pltpu.CompilerParams(use_tc_tiling_on_sc=False,
                                             needs_layout_passes=False),
    )
    def gather_kernel(idx_hbm, table_hbm, out_hbm, table_v, idx_v, rows_v,
                      tsem, isem0, isem1, wsem0, wsem1):
        wid = lax.axis_index("s") * NC + lax.axis_index("c")
        base = wid * PER_W
        obase = base * D
        isems = (isem0, isem1)
        wsems = (wsem0, wsem1)
        lane = lax.iota(jnp.int32, L)
        ltd = lane * D

        tcopy = pltpu.async_copy(table_hbm, table_v, tsem)
        icopies = [
            pltpu.async_copy(idx_hbm.at[pl.ds(base + g * CHUNK, CHUNK)],
                             idx_v.at[g], isems[g])
            for g in range(2)
        ]
        tcopy.wait()

        def gather_chunk(b):
            @plsc.parallel_loop(0, CHUNK // L, unroll=4)
            def body(j):
                iv = idx_v[b, pl.ds(j * L, L)]
                a0 = iv * D
                dst = rows_v.at[b, pl.ds(j * (L * D), L * D)]
                for c in range(D):
                    col = plsc.load_gather(table_v, [a0 + c])
                    plsc.store_scatter(dst, [ltd + c], col)

        for b in range(2):
            icopies[b].wait()
            gather_chunk(b)
            pltpu.async_copy(idx_hbm.at[pl.ds(base + (b + 2) * CHUNK, CHUNK)],
                             idx_v.at[b], isems[b])
            pltpu.async_copy(rows_v.at[b],
                             out_hbm.at[pl.ds(obase + b * RB, RB)],
                             wsems[b])

        def pair(i, carry):
            for b in range(2):
                g = 2 * i + b
                off = base + g * CHUNK
                pltpu.make_async_copy(
                    rows_v.at[b], out_hbm.at[pl.ds(obase + g * RB, RB)],
                    wsems[b]).wait()
                pltpu.make_async_copy(
                    idx_hbm.at[pl.ds(off, CHUNK)], idx_v.at[b],
                    isems[b]).wait()
                gather_chunk(b)
                off2 = base + lax.rem(g + 2, NCHUNK) * CHUNK
                pltpu.async_copy(idx_hbm.at[pl.ds(off2, CHUNK)],
                                 idx_v.at[b], isems[b])
                pltpu.async_copy(rows_v.at[b],
                                 out_hbm.at[pl.ds(obase + g * RB, RB)],
                                 wsems[b])
            return carry

        lax.fori_loop(1, NCHUNK // 2, pair, 0)

        for b in range(2):
            pltpu.make_async_copy(
                rows_v.at[b], out_hbm.at[pl.ds(obase, RB)],
                wsems[b]).wait()
            pltpu.make_async_copy(
                idx_hbm.at[pl.ds(base, CHUNK)], idx_v.at[b],
                isems[b]).wait()

    return gather_kernel


_gather = _make_gather()


def kernel(x, vec_of_char):
    idx = x.reshape(N)
    out = _gather(idx, vec_of_char.reshape(VOCAB * D))
    return out.reshape(B, T, F * D)


# speedup vs baseline: 3.0134x; 2.0992x over previous
import functools

import jax
import jax.numpy as jnp
from jax import lax
from jax.experimental import pallas as pl
from jax.experimental.pallas import tpu as pltpu
from jax.experimental.pallas import tpu_sc as plsc

B, T, F = 1024, 50, 26
VOCAB, D = 1000, 16
NC, NS = 2, 16
NW = NC * NS
B_PER_W = B // NW
L = 16

_DNUMS = lax.GatherDimensionNumbers(
    offset_dims=(), collapsed_slice_dims=(0,), start_index_map=(0,))


def _bcast(iv, l):
    return lax.gather(iv, jnp.full((L, 1), l, jnp.int32), _DNUMS,
                      slice_sizes=(1,),
                      mode=lax.GatherScatterMode.PROMISE_IN_BOUNDS)


def _make_gather():
    mesh = plsc.VectorSubcoreMesh(core_axis_name="c", subcore_axis_name="s")

    @functools.partial(
        pl.kernel,
        mesh=mesh,
        out_type=jax.ShapeDtypeStruct((B, T, F * D), jnp.float32),
        scratch_types=[
            pltpu.VMEM((VOCAB * D,), jnp.float32),
            pltpu.VMEM((T, F), jnp.int32),
            pltpu.VMEM((T, F), jnp.int32),
            pltpu.VMEM((T, F * D), jnp.float32),
            pltpu.VMEM((T, F * D), jnp.float32),
            pltpu.SemaphoreType.DMA,
            pltpu.SemaphoreType.DMA,
            pltpu.SemaphoreType.DMA,
            pltpu.SemaphoreType.DMA,
            pltpu.SemaphoreType.DMA,
        ],
        compiler_params=pltpu.CompilerParams(use_tc_tiling_on_sc=True,
                                             needs_layout_passes=False),
    )
    def gather_kernel(x_hbm, table_hbm, out_hbm, table_v, xin0, xin1,
                      slab0, slab1, tsem, isem0, isem1, wsem0, wsem1):
        wid = lax.axis_index("s") * NC + lax.axis_index("c")
        b0 = wid * B_PER_W
        xins = (xin0, xin1)
        slabs = (slab0, slab1)
        isems = (isem0, isem1)
        wsems = (wsem0, wsem1)
        lane = lax.iota(jnp.int32, L)

        tcopy = pltpu.async_copy(table_hbm, table_v, tsem)
        icopies = [
            pltpu.async_copy(x_hbm.at[b0 + i], xins[i], isems[i])
            for i in range(2)
        ]
        tcopy.wait()

        def gather_slab(xin, slab):
            @plsc.parallel_loop(0, T, unroll=2)
            def row(t):
                iv0 = xin[t, pl.ds(0, L)]
                iv1 = xin[t, pl.ds(F - L, L)]
                for f in range(F):
                    iv, l = (iv0, f) if f < L else (iv1, f - (F - L))
                    a = _bcast(iv, l) * D + lane
                    slab[t, pl.ds(f * D, D)] = plsc.load_gather(table_v, [a])

        for u in range(2):
            icopies[u].wait()
            gather_slab(xins[u], slabs[u])
            pltpu.async_copy(x_hbm.at[b0 + u + 2], xins[u], isems[u])
            pltpu.async_copy(slabs[u], out_hbm.at[b0 + u], wsems[u])

        def pair(i, carry):
            for u in range(2):
                b = 2 * i + u
                pltpu.make_async_copy(
                    slabs[u], out_hbm.at[b0 + b], wsems[u]).wait()
                pltpu.make_async_copy(
                    x_hbm.at[b0 + b], xins[u], isems[u]).wait()
                gather_slab(xins[u], slabs[u])
                nxt = b0 + lax.rem(b + 2, B_PER_W)
                pltpu.async_copy(x_hbm.at[nxt], xins[u], isems[u])
                pltpu.async_copy(slabs[u], out_hbm.at[b0 + b], wsems[u])
            return carry

        lax.fori_loop(1, B_PER_W // 2, pair, 0)

        for u in range(2):
            pltpu.make_async_copy(
                slabs[u], out_hbm.at[b0], wsems[u]).wait()
            pltpu.make_async_copy(
                x_hbm.at[b0], xins[u], isems[u]).wait()

    return gather_kernel


_gather = _make_gather()


def kernel(x, vec_of_char):
    return _gather(x, vec_of_char.reshape(VOCAB * D))


# speedup vs baseline: 4.3810x; 1.4538x over previous
import functools

import jax
import jax.numpy as jnp
from jax import lax
from jax.experimental import pallas as pl
from jax.experimental.pallas import tpu as pltpu
from jax.experimental.pallas import tpu_sc as plsc

B, T, F = 1024, 50, 26
VOCAB, D = 1000, 16
L = 16
FH = F // 2
TH = T // 2
KH = FH * D
NG = 128 // L


def _make_gather():
    mesh = plsc.VectorSubcoreMesh(core_axis_name="c", subcore_axis_name="s")

    @functools.partial(
        pl.kernel,
        mesh=mesh,
        out_type=jax.ShapeDtypeStruct((T, F * D, B), jnp.float32),
        scratch_types=[
            pltpu.VMEM((VOCAB * D,), jnp.float32),
            pltpu.VMEM((FH, 128), jnp.int32),
            pltpu.VMEM((FH, 128), jnp.int32),
            pltpu.VMEM((KH, 128), jnp.float32),
            pltpu.VMEM((KH, 128), jnp.float32),
            pltpu.SemaphoreType.DMA,
            pltpu.SemaphoreType.DMA,
            pltpu.SemaphoreType.DMA,
            pltpu.SemaphoreType.DMA,
            pltpu.SemaphoreType.DMA,
        ],
        compiler_params=pltpu.CompilerParams(use_tc_tiling_on_sc=True,
                                             needs_layout_passes=False),
    )
    def gather_kernel(x_hbm, table_hbm, out_hbm, table_v, xin0, xin1,
                      slab0, slab1, tsem, isem0, isem1, wsem0, wsem1):
        wid = lax.axis_index("s") * 2 + lax.axis_index("c")
        bt = lax.rem(wid, 8)
        kh = lax.rem(wid // 8, 2)
        tg = wid // 16
        bq = bt * 128
        f0 = kh * FH
        k0 = kh * KH
        t0 = tg * TH
        xins = (xin0, xin1)
        slabs = (slab0, slab1)
        isems = (isem0, isem1)
        wsems = (wsem0, wsem1)

        def stage_in(t, u):
            return pltpu.async_copy(
                x_hbm.at[pl.ds(f0, FH), t, pl.ds(bq, 128)], xins[u],
                isems[u])

        def stage_out(t, u):
            return pltpu.async_copy(
                slabs[u], out_hbm.at[t, pl.ds(k0, KH), pl.ds(bq, 128)],
                wsems[u])

        tcopy = pltpu.async_copy(table_hbm, table_v, tsem)
        icopies = [stage_in(t0 + u, u) for u in range(2)]
        tcopy.wait()

        def gather_t(xin, slab):
            @plsc.parallel_loop(0, FH * NG, unroll=2)
            def unit(m):
                f = m >> 3
                g = lax.rem(m, NG)
                a0 = xin[f, pl.ds(g * L, L)] * D
                for c in range(D):
                    slab[f * D + c, pl.ds(g * L, L)] = (
                        plsc.load_gather(table_v, [a0 + c]))

        for u in range(2):
            icopies[u].wait()
            gather_t(xins[u], slabs[u])
            stage_in(t0 + u + 2, u)
            stage_out(t0 + u, u)

        def pair(i, carry):
            for u in range(2):
                t = t0 + 2 * i + u
                pltpu.make_async_copy(
                    slabs[u], out_hbm.at[t, pl.ds(k0, KH), pl.ds(bq, 128)],
                    wsems[u]).wait()
                pltpu.make_async_copy(
                    x_hbm.at[pl.ds(f0, FH), t, pl.ds(bq, 128)], xins[u],
                    isems[u]).wait()
                gather_t(xins[u], slabs[u])
                stage_in(t0 + lax.rem(2 * i + u + 2, TH), u)
                stage_out(t, u)
            return carry

        lax.fori_loop(1, TH // 2, pair, 0)

        tl = t0 + TH - 1
        pltpu.make_async_copy(
            slabs[0], out_hbm.at[tl, pl.ds(k0, KH), pl.ds(bq, 128)],
            wsems[0]).wait()
        pltpu.make_async_copy(
            x_hbm.at[pl.ds(f0, FH), tl, pl.ds(bq, 128)], xins[0],
            isems[0]).wait()
        gather_t(xins[0], slabs[0])
        stage_out(tl, 0)

        for u in range(2):
            pltpu.make_async_copy(
                slabs[u], out_hbm.at[t0, pl.ds(k0, KH), pl.ds(bq, 128)],
                wsems[u]).wait()
        pltpu.make_async_copy(
            x_hbm.at[pl.ds(f0, FH), t0, pl.ds(bq, 128)], xins[1],
            isems[1]).wait()

    return gather_kernel


_gather = _make_gather()


def kernel(x, vec_of_char):
    xt = jnp.transpose(x, (2, 1, 0))
    out_t = _gather(xt, vec_of_char.reshape(VOCAB * D))
    return jnp.transpose(out_t, (2, 0, 1))
